# baseline (speedup 1.0000x reference)
import functools

import jax
import jax.numpy as jnp
from jax import lax
from jax.experimental import pallas as pl
from jax.experimental.pallas import tpu as pltpu
from jax.experimental.pallas import tpu_sc as plsc

TILE = 256


def _router_kernel(x_ref, g_ref, idx_ref, w_ref, *, n_exp):
    logits = jnp.dot(x_ref[...], g_ref[...], preferred_element_type=jnp.float32)
    eidx = jax.lax.broadcasted_iota(jnp.int32, logits.shape, 1)
    m1 = jnp.max(logits, axis=1, keepdims=True)
    i1 = jnp.min(jnp.where(logits == m1, eidx, n_exp), axis=1, keepdims=True)
    masked = jnp.where(eidx == i1, -jnp.inf, logits)
    m2 = jnp.max(masked, axis=1, keepdims=True)
    i2 = jnp.min(jnp.where(masked == m2, eidx, n_exp), axis=1, keepdims=True)
    e2 = jnp.exp(m2 - m1)
    w1 = 1.0 / (1.0 + e2)
    w2 = e2 / (1.0 + e2)
    idx_ref[...] = jnp.concatenate([i1, i2], axis=1)
    w_ref[...] = jnp.concatenate([w1, w2], axis=1)


def _gmm_kernel(grp_ref, xs_ref, wi0_ref, wi1_ref, wo_ref, w_ref, outs_ref):
    xs = xs_ref[...].astype(jnp.bfloat16)
    a0 = jnp.dot(xs, wi0_ref[0], preferred_element_type=jnp.float32)
    a1 = jnp.dot(xs, wi1_ref[0], preferred_element_type=jnp.float32)
    h = (jax.nn.gelu(a0) * a1).astype(jnp.bfloat16)
    outs_ref[...] = jnp.dot(h, wo_ref[0],
                            preferred_element_type=jnp.float32) * w_ref[...]


def _sc_gather_rows(x, idx, padrows, d):
    info = plsc.get_sparse_core_info()
    nw = info.num_cores * info.num_subcores
    rows_per_w = padrows // nw
    chunk = 48
    n_chunks = rows_per_w // chunk
    mesh = plsc.VectorSubcoreMesh(core_axis_name="c", subcore_axis_name="s")

    @functools.partial(
        pl.kernel, mesh=mesh,
        out_type=jax.ShapeDtypeStruct((padrows, d), jnp.float32),
        scratch_types=[
            pltpu.VMEM((rows_per_w,), jnp.int32),
            pltpu.VMEM((chunk, d), jnp.float32),
            pltpu.VMEM((chunk, d), jnp.float32),
            pltpu.SemaphoreType.DMA,
            pltpu.SemaphoreType.DMA,
            pltpu.SemaphoreType.DMA,
            pltpu.SemaphoreType.DMA,
        ],
    )
    def k(x_hbm, idx_hbm, xs_hbm, idx_v, rows0_v, rows1_v,
          g0_sem, g1_sem, w0_sem, w1_sem):
        wid = lax.axis_index("s") * info.num_cores + lax.axis_index("c")
        base = wid * rows_per_w
        pltpu.sync_copy(idx_hbm.at[pl.ds(base, rows_per_w)], idx_v)
        rows = (rows0_v, rows1_v)
        gsem = (g0_sem, g1_sem)
        wsem = (w0_sem, w1_sem)
        g = [None, None]
        wb = [None, None]
        for c in range(n_chunks):
            bi = c & 1
            if wb[bi] is not None:
                wb[bi].wait()
            g[bi] = pltpu.async_copy(
                x_hbm.at[idx_v.at[pl.ds(c * chunk, chunk)]], rows[bi], gsem[bi])
            if c >= 1:
                pb = (c - 1) & 1
                g[pb].wait()
                wb[pb] = pltpu.async_copy(
                    rows[pb], xs_hbm.at[pl.ds(base + (c - 1) * chunk, chunk)],
                    wsem[pb])
        lb = (n_chunks - 1) & 1
        g[lb].wait()
        wb[lb] = pltpu.async_copy(
            rows[lb], xs_hbm.at[pl.ds(base + (n_chunks - 1) * chunk, chunk)],
            wsem[lb])
        wb[0].wait()
        wb[1].wait()

    return k(x, idx)


def _sc_combine_rows(outs, pos0, pos1, t, d):
    info = plsc.get_sparse_core_info()
    nw = info.num_cores * info.num_subcores
    tok_per_w = t // nw
    chunk = 32
    n_chunks = tok_per_w // chunk
    lanes = info.num_lanes
    mesh = plsc.VectorSubcoreMesh(core_axis_name="c", subcore_axis_name="s")

    @functools.partial(
        pl.kernel, mesh=mesh,
        out_type=jax.ShapeDtypeStruct((t, d), jnp.float32),
        scratch_types=[
            pltpu.VMEM((chunk,), jnp.int32),
            pltpu.VMEM((chunk,), jnp.int32),
            pltpu.VMEM((chunk, d), jnp.float32),
            pltpu.VMEM((chunk, d), jnp.float32),
            pltpu.SemaphoreType.DMA,
            pltpu.SemaphoreType.DMA,
        ],
    )
    def k(outs_hbm, pos0_hbm, pos1_hbm, out_hbm,
          idx0_v, idx1_v, rows0_v, rows1_v, sem0, sem1):
        wid = lax.axis_index("s") * info.num_cores + lax.axis_index("c")
        base = wid * tok_per_w
        slices_per_row = d // lanes

        def body(c, _):
            off = base + c * chunk
            pltpu.sync_copy(pos0_hbm.at[pl.ds(off, chunk)], idx0_v)
            pltpu.sync_copy(pos1_hbm.at[pl.ds(off, chunk)], idx1_v)
            cp0 = pltpu.async_copy(outs_hbm.at[idx0_v], rows0_v, sem0)
            cp1 = pltpu.async_copy(outs_hbm.at[idx1_v], rows1_v, sem1)
            cp0.wait()
            cp1.wait()

            def add_row(r, _):
                for s in range(slices_per_row):
                    sl = pl.ds(s * lanes, lanes)
                    rows0_v[r, sl] = rows0_v[r, sl] + rows1_v[r, sl]
                return 0
            lax.fori_loop(0, chunk, add_row, 0)
            pltpu.sync_copy(rows0_v, out_hbm.at[pl.ds(off, chunk)])
            return 0
        lax.fori_loop(0, n_chunks, body, 0)

    return k(outs, pos0, pos1)


def kernel(inputs, decoder_segment_ids, decoder_positions, gate_kernel, wi_0, wi_1, wo):
    del decoder_segment_ids, decoder_positions
    b, s, d = inputs.shape
    t = b * s
    n_exp = gate_kernel.shape[-1]
    f_dim = wi_0.shape[-1]
    k = 2
    nt = (t * k) // TILE + n_exp
    padrows = nt * TILE

    x = inputs.reshape(t, d)

    top_idx, top_w = pl.pallas_call(
        functools.partial(_router_kernel, n_exp=n_exp),
        out_shape=(
            jax.ShapeDtypeStruct((t, k), jnp.int32),
            jax.ShapeDtypeStruct((t, k), jnp.float32),
        ),
    )(x, gate_kernel)

    flat_e = top_idx.reshape(-1)
    flat_t = (jnp.arange(t * k, dtype=jnp.int32) // k)
    flat_w = top_w.reshape(-1)
    counts = jnp.bincount(flat_e, length=n_exp)
    padded = ((counts + TILE - 1) // TILE) * TILE
    pend = jnp.cumsum(padded)
    pstart = pend - padded
    ustart = jnp.cumsum(counts) - counts
    order = jnp.argsort(flat_e, stable=True)
    se = flat_e[order]
    pos = jnp.arange(t * k)
    dest = pstart[se] + (pos - ustart[se])
    sorted_tok = jnp.zeros(padrows, jnp.int32).at[dest].set(flat_t[order])
    sorted_w = jnp.zeros(padrows, jnp.float32).at[dest].set(flat_w[order])
    rowpos = jnp.zeros(t * k, jnp.int32).at[order].set(
        dest.astype(jnp.int32)).reshape(t, k)
    tile_grp = jnp.clip(
        jnp.searchsorted(pend, jnp.arange(nt) * TILE, side='right'),
        0, n_exp - 1).astype(jnp.int32)

    xs = _sc_gather_rows(x, sorted_tok, padrows, d)

    grid_spec = pltpu.PrefetchScalarGridSpec(
        num_scalar_prefetch=1,
        grid=(nt,),
        in_specs=[
            pl.BlockSpec((TILE, d), lambda i, grp: (i, 0)),
            pl.BlockSpec((1, d, f_dim), lambda i, grp: (grp[i], 0, 0)),
            pl.BlockSpec((1, d, f_dim), lambda i, grp: (grp[i], 0, 0)),
            pl.BlockSpec((1, f_dim, d), lambda i, grp: (grp[i], 0, 0)),
            pl.BlockSpec((TILE, 1), lambda i, grp: (i, 0)),
        ],
        out_specs=pl.BlockSpec((TILE, d), lambda i, grp: (i, 0)),
    )
    outs = pl.pallas_call(
        _gmm_kernel,
        grid_spec=grid_spec,
        out_shape=jax.ShapeDtypeStruct((padrows, d), jnp.float32),
        compiler_params=pltpu.CompilerParams(
            dimension_semantics=("arbitrary",),
            vmem_limit_bytes=100 * 1024 * 1024,
        ),
    )(tile_grp, xs,
      wi_0.astype(jnp.bfloat16), wi_1.astype(jnp.bfloat16),
      wo.astype(jnp.bfloat16), sorted_w.reshape(padrows, 1))

    out = _sc_combine_rows(outs, rowpos[:, 0], rowpos[:, 1], t, d)
    return out.reshape(b, s, d)


# speedup vs baseline: 1.0707x; 1.0707x over previous
import functools

import jax
import jax.numpy as jnp
from jax import lax
from jax.experimental import pallas as pl
from jax.experimental.pallas import tpu as pltpu
from jax.experimental.pallas import tpu_sc as plsc

TILE = 256


def _router_kernel(x_ref, g_ref, idx_ref, w_ref, *, n_exp):
    logits = jnp.dot(x_ref[...], g_ref[...], preferred_element_type=jnp.float32)
    eidx = jax.lax.broadcasted_iota(jnp.int32, logits.shape, 1)
    m1 = jnp.max(logits, axis=1, keepdims=True)
    i1 = jnp.min(jnp.where(logits == m1, eidx, n_exp), axis=1, keepdims=True)
    masked = jnp.where(eidx == i1, -jnp.inf, logits)
    m2 = jnp.max(masked, axis=1, keepdims=True)
    i2 = jnp.min(jnp.where(masked == m2, eidx, n_exp), axis=1, keepdims=True)
    e2 = jnp.exp(m2 - m1)
    w1 = 1.0 / (1.0 + e2)
    w2 = e2 / (1.0 + e2)
    idx_ref[...] = jnp.concatenate([i1, i2], axis=1)
    w_ref[...] = jnp.concatenate([w1, w2], axis=1)


def _gmm_kernel(grp_ref, xs_ref, wi0_ref, wi1_ref, wo_ref, w_ref, outs_ref,
                acc_scr, *, nf):
    f = pl.program_id(1)
    xs = xs_ref[...]
    a0 = jnp.dot(xs, wi0_ref[0], preferred_element_type=jnp.float32)
    a1 = jnp.dot(xs, wi1_ref[0], preferred_element_type=jnp.float32)
    h = jax.nn.gelu(a0) * a1
    contrib = jnp.dot(h, wo_ref[0], preferred_element_type=jnp.float32)

    @pl.when(f == 0)
    def _():
        acc_scr[...] = contrib

    @pl.when(f != 0)
    def _():
        acc_scr[...] += contrib

    @pl.when(f == nf - 1)
    def _():
        outs_ref[...] = acc_scr[...] * w_ref[...]


def _sc_gather_rows(x, idx, padrows, d):
    info = plsc.get_sparse_core_info()
    nw = info.num_cores * info.num_subcores
    rows_per_w = padrows // nw
    chunk = 48
    n_chunks = rows_per_w // chunk
    mesh = plsc.VectorSubcoreMesh(core_axis_name="c", subcore_axis_name="s")

    @functools.partial(
        pl.kernel, mesh=mesh,
        out_type=jax.ShapeDtypeStruct((padrows, d), jnp.float32),
        scratch_types=[
            pltpu.VMEM((rows_per_w,), jnp.int32),
            pltpu.VMEM((chunk, d), jnp.float32),
            pltpu.VMEM((chunk, d), jnp.float32),
            pltpu.SemaphoreType.DMA,
            pltpu.SemaphoreType.DMA,
            pltpu.SemaphoreType.DMA,
            pltpu.SemaphoreType.DMA,
        ],
    )
    def k(x_hbm, idx_hbm, xs_hbm, idx_v, rows0_v, rows1_v,
          g0_sem, g1_sem, w0_sem, w1_sem):
        wid = lax.axis_index("s") * info.num_cores + lax.axis_index("c")
        base = wid * rows_per_w
        pltpu.sync_copy(idx_hbm.at[pl.ds(base, rows_per_w)], idx_v)
        rows = (rows0_v, rows1_v)
        gsem = (g0_sem, g1_sem)
        wsem = (w0_sem, w1_sem)
        g = [None, None]
        wb = [None, None]
        for c in range(n_chunks):
            bi = c & 1
            if wb[bi] is not None:
                wb[bi].wait()
            g[bi] = pltpu.async_copy(
                x_hbm.at[idx_v.at[pl.ds(c * chunk, chunk)]], rows[bi], gsem[bi])
            if c >= 1:
                pb = (c - 1) & 1
                g[pb].wait()
                wb[pb] = pltpu.async_copy(
                    rows[pb], xs_hbm.at[pl.ds(base + (c - 1) * chunk, chunk)],
                    wsem[pb])
        lb = (n_chunks - 1) & 1
        g[lb].wait()
        wb[lb] = pltpu.async_copy(
            rows[lb], xs_hbm.at[pl.ds(base + (n_chunks - 1) * chunk, chunk)],
            wsem[lb])
        wb[0].wait()
        wb[1].wait()

    return k(x, idx)


def _sc_combine_rows(outs, pos0, pos1, t, d):
    info = plsc.get_sparse_core_info()
    nw = info.num_cores * info.num_subcores
    tok_per_w = t // nw
    chunk = 32
    n_chunks = tok_per_w // chunk
    lanes = info.num_lanes
    mesh = plsc.VectorSubcoreMesh(core_axis_name="c", subcore_axis_name="s")

    @functools.partial(
        pl.kernel, mesh=mesh,
        out_type=jax.ShapeDtypeStruct((t, d), jnp.float32),
        scratch_types=[
            pltpu.VMEM((chunk,), jnp.int32),
            pltpu.VMEM((chunk,), jnp.int32),
            pltpu.VMEM((chunk, d), jnp.float32),
            pltpu.VMEM((chunk, d), jnp.float32),
            pltpu.SemaphoreType.DMA,
            pltpu.SemaphoreType.DMA,
        ],
    )
    def k(outs_hbm, pos0_hbm, pos1_hbm, out_hbm,
          idx0_v, idx1_v, rows0_v, rows1_v, sem0, sem1):
        wid = lax.axis_index("s") * info.num_cores + lax.axis_index("c")
        base = wid * tok_per_w
        slices_per_row = d // lanes

        def body(c, _):
            off = base + c * chunk
            pltpu.sync_copy(pos0_hbm.at[pl.ds(off, chunk)], idx0_v)
            pltpu.sync_copy(pos1_hbm.at[pl.ds(off, chunk)], idx1_v)
            cp0 = pltpu.async_copy(outs_hbm.at[idx0_v], rows0_v, sem0)
            cp1 = pltpu.async_copy(outs_hbm.at[idx1_v], rows1_v, sem1)
            cp0.wait()
            cp1.wait()

            def add_row(r, _):
                for s in range(slices_per_row):
                    sl = pl.ds(s * lanes, lanes)
                    rows0_v[r, sl] = rows0_v[r, sl] + rows1_v[r, sl]
                return 0
            lax.fori_loop(0, chunk, add_row, 0)
            pltpu.sync_copy(rows0_v, out_hbm.at[pl.ds(off, chunk)])
            return 0
        lax.fori_loop(0, n_chunks, body, 0)

    return k(outs, pos0, pos1)


def kernel(inputs, decoder_segment_ids, decoder_positions, gate_kernel, wi_0, wi_1, wo):
    del decoder_segment_ids, decoder_positions
    b, s, d = inputs.shape
    t = b * s
    n_exp = gate_kernel.shape[-1]
    f_dim = wi_0.shape[-1]
    k = 2
    nt = (t * k) // TILE + n_exp
    padrows = nt * TILE

    x = inputs.reshape(t, d)

    top_idx, top_w = pl.pallas_call(
        functools.partial(_router_kernel, n_exp=n_exp),
        out_shape=(
            jax.ShapeDtypeStruct((t, k), jnp.int32),
            jax.ShapeDtypeStruct((t, k), jnp.float32),
        ),
    )(x, gate_kernel)

    flat_e = top_idx.reshape(-1)
    flat_t = (jnp.arange(t * k, dtype=jnp.int32) // k)
    flat_w = top_w.reshape(-1)
    counts = jnp.bincount(flat_e, length=n_exp)
    padded = ((counts + TILE - 1) // TILE) * TILE
    pend = jnp.cumsum(padded)
    pstart = pend - padded
    ustart = jnp.cumsum(counts) - counts
    order = jnp.argsort(flat_e, stable=True)
    se = flat_e[order]
    pos = jnp.arange(t * k)
    dest = pstart[se] + (pos - ustart[se])
    sorted_tok = jnp.zeros(padrows, jnp.int32).at[dest].set(flat_t[order])
    sorted_w = jnp.zeros(padrows, jnp.float32).at[dest].set(flat_w[order])
    rowpos = jnp.zeros(t * k, jnp.int32).at[order].set(
        dest.astype(jnp.int32)).reshape(t, k)
    tile_grp = jnp.clip(
        jnp.searchsorted(pend, jnp.arange(nt) * TILE, side='right'),
        0, n_exp - 1).astype(jnp.int32)

    xs = _sc_gather_rows(x, sorted_tok, padrows, d)

    nf = 2
    fb = f_dim // nf
    grid_spec = pltpu.PrefetchScalarGridSpec(
        num_scalar_prefetch=1,
        grid=(nt, nf),
        in_specs=[
            pl.BlockSpec((TILE, d), lambda i, f, grp: (i, 0)),
            pl.BlockSpec((1, d, fb), lambda i, f, grp: (grp[i], 0, f)),
            pl.BlockSpec((1, d, fb), lambda i, f, grp: (grp[i], 0, f)),
            pl.BlockSpec((1, fb, d), lambda i, f, grp: (grp[i], f, 0)),
            pl.BlockSpec((TILE, 1), lambda i, f, grp: (i, 0)),
        ],
        out_specs=pl.BlockSpec((TILE, d), lambda i, f, grp: (i, 0)),
        scratch_shapes=[pltpu.VMEM((TILE, d), jnp.float32)],
    )
    outs = pl.pallas_call(
        functools.partial(_gmm_kernel, nf=nf),
        grid_spec=grid_spec,
        out_shape=jax.ShapeDtypeStruct((padrows, d), jnp.float32),
        compiler_params=pltpu.CompilerParams(
            dimension_semantics=("arbitrary", "arbitrary"),
            vmem_limit_bytes=100 * 1024 * 1024,
        ),
    )(tile_grp, xs, wi_0, wi_1, wo, sorted_w.reshape(padrows, 1))

    out = _sc_combine_rows(outs, rowpos[:, 0], rowpos[:, 1], t, d)
    return out.reshape(b, s, d)


# speedup vs baseline: 1.1536x; 1.0775x over previous
import functools

import jax
import jax.numpy as jnp
from jax import lax
from jax.experimental import pallas as pl
from jax.experimental.pallas import tpu as pltpu
from jax.experimental.pallas import tpu_sc as plsc

TILE = 256


def _router_kernel(x_ref, g_ref, idx_ref, w_ref, *, n_exp):
    logits = jnp.dot(x_ref[...], g_ref[...], preferred_element_type=jnp.float32)
    eidx = jax.lax.broadcasted_iota(jnp.int32, logits.shape, 1)
    m1 = jnp.max(logits, axis=1, keepdims=True)
    i1 = jnp.min(jnp.where(logits == m1, eidx, n_exp), axis=1, keepdims=True)
    masked = jnp.where(eidx == i1, -jnp.inf, logits)
    m2 = jnp.max(masked, axis=1, keepdims=True)
    i2 = jnp.min(jnp.where(masked == m2, eidx, n_exp), axis=1, keepdims=True)
    e2 = jnp.exp(m2 - m1)
    w1 = 1.0 / (1.0 + e2)
    w2 = e2 / (1.0 + e2)
    idx_ref[...] = jnp.concatenate([i1, i2], axis=1)
    w_ref[...] = jnp.concatenate([w1, w2], axis=1)


def _mlp_half(xs, wi0, wi1, wo):
    a0 = jnp.dot(xs, wi0, preferred_element_type=jnp.float32)
    a1 = jnp.dot(xs, wi1, preferred_element_type=jnp.float32)
    h = jax.nn.gelu(a0) * a1
    return jnp.dot(h, wo, preferred_element_type=jnp.float32)


def _gmm_a_kernel(grp_ref, xs_ref, wi0_ref, wi1_ref, wo_ref, w_ref, outs_ref,
                  *, nt):
    @pl.when(pl.program_id(0) < grp_ref[nt])
    def _():
        outs_ref[...] = _mlp_half(
            xs_ref[...], wi0_ref[0], wi1_ref[0], wo_ref[0]) * w_ref[...]


def _gmm_b_kernel(grp_ref, xs_ref, wi0_ref, wi1_ref, wo_ref, w_ref, prev_ref,
                  outs_ref, *, nt):
    @pl.when(pl.program_id(0) < grp_ref[nt])
    def _():
        outs_ref[...] = prev_ref[...] + _mlp_half(
            xs_ref[...], wi0_ref[0], wi1_ref[0], wo_ref[0]) * w_ref[...]


def _sc_gather_rows(x, idx, padrows, d):
    info = plsc.get_sparse_core_info()
    nw = info.num_cores * info.num_subcores
    rows_per_w = padrows // nw
    nbuf = 4
    chunk = 24
    n_chunks = rows_per_w // chunk
    mesh = plsc.VectorSubcoreMesh(core_axis_name="c", subcore_axis_name="s")

    @functools.partial(
        pl.kernel, mesh=mesh,
        out_type=jax.ShapeDtypeStruct((padrows, d), jnp.float32),
        scratch_types=[pltpu.VMEM((rows_per_w,), jnp.int32)]
        + [pltpu.VMEM((chunk, d), jnp.float32)] * nbuf
        + [pltpu.SemaphoreType.DMA] * (2 * nbuf),
    )
    def k(x_hbm, idx_hbm, xs_hbm, idx_v, *bufs_sems):
        rows = bufs_sems[:nbuf]
        gsem = bufs_sems[nbuf:2 * nbuf]
        wsem = bufs_sems[2 * nbuf:]
        wid = lax.axis_index("s") * info.num_cores + lax.axis_index("c")
        base = wid * rows_per_w
        pltpu.sync_copy(idx_hbm.at[pl.ds(base, rows_per_w)], idx_v)
        g = [None] * nbuf
        wb = [None] * nbuf
        for c in range(n_chunks):
            b = c % nbuf
            if wb[b] is not None:
                wb[b].wait()
            g[b] = pltpu.async_copy(
                x_hbm.at[idx_v.at[pl.ds(c * chunk, chunk)]], rows[b], gsem[b])
            if c >= nbuf - 1:
                oc = c - (nbuf - 1)
                ob = oc % nbuf
                g[ob].wait()
                wb[ob] = pltpu.async_copy(
                    rows[ob], xs_hbm.at[pl.ds(base + oc * chunk, chunk)],
                    wsem[ob])
        for oc in range(n_chunks - (nbuf - 1), n_chunks):
            ob = oc % nbuf
            g[ob].wait()
            wb[ob] = pltpu.async_copy(
                rows[ob], xs_hbm.at[pl.ds(base + oc * chunk, chunk)], wsem[ob])
        for b in range(nbuf):
            if wb[b] is not None:
                wb[b].wait()

    return k(x, idx)


def _sc_combine_rows(outs, pos0, pos1, t, d):
    info = plsc.get_sparse_core_info()
    nw = info.num_cores * info.num_subcores
    tok_per_w = t // nw
    chunk = 32
    n_chunks = tok_per_w // chunk
    lanes = info.num_lanes
    mesh = plsc.VectorSubcoreMesh(core_axis_name="c", subcore_axis_name="s")

    @functools.partial(
        pl.kernel, mesh=mesh,
        out_type=jax.ShapeDtypeStruct((t, d), jnp.float32),
        scratch_types=[
            pltpu.VMEM((chunk,), jnp.int32),
            pltpu.VMEM((chunk,), jnp.int32),
            pltpu.VMEM((chunk, d), jnp.float32),
            pltpu.VMEM((chunk, d), jnp.float32),
            pltpu.SemaphoreType.DMA,
            pltpu.SemaphoreType.DMA,
        ],
    )
    def k(outs_hbm, pos0_hbm, pos1_hbm, out_hbm,
          idx0_v, idx1_v, rows0_v, rows1_v, sem0, sem1):
        wid = lax.axis_index("s") * info.num_cores + lax.axis_index("c")
        base = wid * tok_per_w
        slices_per_row = d // lanes

        def body(c, _):
            off = base + c * chunk
            pltpu.sync_copy(pos0_hbm.at[pl.ds(off, chunk)], idx0_v)
            pltpu.sync_copy(pos1_hbm.at[pl.ds(off, chunk)], idx1_v)
            cp0 = pltpu.async_copy(outs_hbm.at[idx0_v], rows0_v, sem0)
            cp1 = pltpu.async_copy(outs_hbm.at[idx1_v], rows1_v, sem1)
            cp0.wait()
            cp1.wait()

            def add_row(r, _):
                for s in range(slices_per_row):
                    sl = pl.ds(s * lanes, lanes)
                    rows0_v[r, sl] = rows0_v[r, sl] + rows1_v[r, sl]
                return 0
            lax.fori_loop(0, chunk, add_row, 0)
            pltpu.sync_copy(rows0_v, out_hbm.at[pl.ds(off, chunk)])
            return 0
        lax.fori_loop(0, n_chunks, body, 0)

    return k(outs, pos0, pos1)


def kernel(inputs, decoder_segment_ids, decoder_positions, gate_kernel, wi_0, wi_1, wo):
    del decoder_segment_ids, decoder_positions
    b, s, d = inputs.shape
    t = b * s
    n_exp = gate_kernel.shape[-1]
    f_dim = wi_0.shape[-1]
    k = 2
    tk = t * k
    nt = tk // TILE + n_exp
    padrows = nt * TILE
    fb = f_dim // 2

    x = inputs.reshape(t, d)

    top_idx, top_w = pl.pallas_call(
        functools.partial(_router_kernel, n_exp=n_exp),
        out_shape=(
            jax.ShapeDtypeStruct((t, k), jnp.int32),
            jax.ShapeDtypeStruct((t, k), jnp.float32),
        ),
    )(x, gate_kernel)

    flat_e = top_idx.reshape(-1)
    flat_t = (jnp.arange(tk, dtype=jnp.int32) // k)
    flat_w = top_w.reshape(-1)
    counts = jnp.bincount(flat_e, length=n_exp)
    padded = ((counts + TILE - 1) // TILE) * TILE
    pend = jnp.cumsum(padded)
    pstart = pend - padded
    ustart = jnp.cumsum(counts) - counts
    order = jnp.argsort(flat_e, stable=True)
    inv = jnp.argsort(order)
    se = flat_e[order]
    pos = jnp.arange(tk)
    dest = pstart[se] + (pos - ustart[se])
    rowpos = dest[inv].astype(jnp.int32).reshape(t, k)
    prow = jnp.arange(padrows)
    g_row = jnp.minimum(
        jnp.searchsorted(pend, prow, side='right'), n_exp - 1)
    sp = prow - pstart[g_row] + ustart[g_row]
    valid = (prow - pstart[g_row]) < counts[g_row]
    slot = order[jnp.clip(sp, 0, tk - 1)]
    sorted_tok = jnp.where(valid, flat_t[slot], 0).astype(jnp.int32)
    sorted_w = jnp.where(valid, flat_w[slot], 0.0)
    tile_grp = jnp.clip(
        jnp.searchsorted(pend, jnp.arange(nt) * TILE, side='right'),
        0, n_exp - 1).astype(jnp.int32)
    n_active = ((pend[-1] + TILE - 1) // TILE).astype(jnp.int32)
    grp_arr = jnp.concatenate([tile_grp, n_active[None]])

    xs = _sc_gather_rows(x, sorted_tok, padrows, d)

    sw = sorted_w.reshape(padrows, 1)
    common = dict(
        out_shape=jax.ShapeDtypeStruct((padrows, d), jnp.float32),
        compiler_params=pltpu.CompilerParams(
            dimension_semantics=("arbitrary",),
            vmem_limit_bytes=100 * 1024 * 1024,
        ),
    )
    xs_spec = pl.BlockSpec((TILE, d), lambda i, grp: (i, 0))
    w_spec = pl.BlockSpec((TILE, 1), lambda i, grp: (i, 0))

    def wi_spec(half):
        return pl.BlockSpec((1, d, fb), lambda i, grp: (grp[i], 0, half))

    def wo_spec(half):
        return pl.BlockSpec((1, fb, d), lambda i, grp: (grp[i], half, 0))

    outs_a = pl.pallas_call(
        functools.partial(_gmm_a_kernel, nt=nt),
        grid_spec=pltpu.PrefetchScalarGridSpec(
            num_scalar_prefetch=1,
            grid=(nt,),
            in_specs=[xs_spec, wi_spec(0), wi_spec(0), wo_spec(0), w_spec],
            out_specs=pl.BlockSpec((TILE, d), lambda i, grp: (i, 0)),
        ),
        **common,
    )(grp_arr, xs, wi_0, wi_1, wo, sw)

    outs = pl.pallas_call(
        functools.partial(_gmm_b_kernel, nt=nt),
        grid_spec=pltpu.PrefetchScalarGridSpec(
            num_scalar_prefetch=1,
            grid=(nt,),
            in_specs=[xs_spec, wi_spec(1), wi_spec(1), wo_spec(1), w_spec,
                      pl.BlockSpec((TILE, d), lambda i, grp: (i, 0))],
            out_specs=pl.BlockSpec((TILE, d), lambda i, grp: (i, 0)),
        ),
        **common,
    )(grp_arr, xs, wi_0, wi_1, wo, sw, outs_a)

    out = _sc_combine_rows(outs, rowpos[:, 0], rowpos[:, 1], t, d)
    return out.reshape(b, s, d)
